# baseline (speedup 1.0000x reference)
import functools

import jax
import jax.numpy as jnp
from jax import lax
from jax.experimental import pallas as pl
from jax.experimental.pallas import tpu as pltpu
from jax.experimental.pallas import tpu_sc as plsc

N_ROWS = 128
ROW = 32768
L = 16
NVEC = ROW // L
NUM_CORES = 2
NUM_SUBCORES = 16
WORKERS = NUM_CORES * NUM_SUBCORES
ROWS_PER_W = N_ROWS // WORKERS

B2, G2, V2, SH2 = 512, 32, 64, 10
B3, G3, V3, SH3 = 256, 64, 32, 9


def _clear(ref, n, unroll=16):
    zeros = jnp.zeros((L,), jnp.int32)

    @plsc.parallel_loop(0, n // L, unroll=unroll)
    def _(i):
        ref[pl.ds(i * L, L)] = zeros


def _excl_prefix(hist, nvec, unroll=8):
    @plsc.parallel_loop(0, nvec, unroll=unroll, carry=jnp.int32(0))
    def _(i, carry):
        h = hist[pl.ds(i * L, L)]
        inc = plsc.cumsum(h)
        hist[pl.ds(i * L, L)] = inc - h + carry
        return carry + jnp.sum(h)


def _block_offsets(ch, nb, g_blocks, unroll=2):
    @plsc.parallel_loop(0, nb // L, unroll=unroll, carry=jnp.int32(0))
    def _(v, carry):
        t = [ch[pl.ds(g * nb + v * L, L)] for g in range(g_blocks)]
        tot = t[0]
        for g in range(1, g_blocks):
            tot = tot + t[g]
        inc = plsc.cumsum(tot)
        run = inc - tot + carry
        ch[pl.ds(0 * nb + v * L, L)] = run
        for g in range(1, g_blocks):
            run = run + t[g - 1]
            ch[pl.ds(g * nb + v * L, L)] = run
        return carry + jnp.sum(tot)


def _sc_body(in_hbm, out_hbm, buf_a, buf_b, hist1, ch2):
    cid = lax.axis_index("c")
    sid = lax.axis_index("s")
    wid = sid * NUM_CORES + cid
    lane = lax.iota(jnp.int32, L)
    ones = jnp.ones((L,), jnp.int32)

    def do_row(j, c0):
        r = wid * ROWS_PER_W + j
        pltpu.sync_copy(in_hbm.at[r], buf_a)

        _clear(hist1, ROW)

        @plsc.parallel_loop(0, NVEC, unroll=4)
        def _(i):
            u = plsc.bitcast(buf_a[pl.ds(i * L, L)], jnp.int32)
            m = lax.shift_right_arithmetic(u, 31)
            k = u ^ ((m ^ jnp.int32(-1)) & jnp.int32(0x7FFFFFFF))
            buf_a[pl.ds(i * L, L)] = plsc.bitcast(k, jnp.float32)
            plsc.addupdate_scatter(hist1, [k & jnp.int32(0x7FFF)], ones)

        _excl_prefix(hist1, NVEC)
        _clear(ch2, G2 * B2)

        def stage(i):
            k = plsc.bitcast(buf_a[pl.ds(i * L, L)], jnp.int32)
            d = k & jnp.int32(0x7FFF)
            cnt, _ = plsc.scan_count(d)
            p = (k & jnp.int32(-32768)) | (i * L + lane)
            return d, cnt, p

        def commit(d, cnt, p):
            pos = plsc.load_gather(hist1, [d]) + cnt - 1
            plsc.store_scatter(buf_b, [pos], p)
            plsc.addupdate_scatter(hist1, [d], ones)
            d2 = lax.shift_right_logical(p, 15) & jnp.int32(B2 - 1)
            blk = lax.shift_right_logical(pos, SH2)
            plsc.addupdate_scatter(ch2, [blk * B2 + d2], ones)

        def body(i, carry):
            nxt = stage(i + 1)
            commit(*carry)
            return nxt

        last = lax.fori_loop(0, NVEC - 1, body, stage(0), unroll=4)
        commit(*last)

        _block_offsets(ch2, B2, G2)
        _clear(hist1, G3 * B3)

        @plsc.parallel_loop(0, G2, unroll=2)
        def _(g):
            for v in range(V2):
                i = g * V2 + v
                p = buf_b[pl.ds(i * L, L)]
                d = lax.shift_right_logical(p, 15) & jnp.int32(B2 - 1)
                cnt, _ = plsc.scan_count(d)
                pos = plsc.load_gather(ch2, [g * B2 + d]) + cnt - 1
                plsc.addupdate_scatter(ch2, [g * B2 + d], ones)
                plsc.store_scatter(buf_a, [pos],
                                   plsc.bitcast(p, jnp.float32))
                d3 = lax.shift_right_logical(p, 24) & jnp.int32(B3 - 1)
                blk = lax.shift_right_logical(pos, SH3)
                plsc.addupdate_scatter(hist1, [blk * B3 + d3], ones)

        _block_offsets(hist1, B3, G3)

        @plsc.parallel_loop(0, G3, unroll=2)
        def _(g):
            for v in range(V3):
                i = g * V3 + v
                p = plsc.bitcast(buf_a[pl.ds(i * L, L)], jnp.int32)
                d = lax.shift_right_logical(p, 24) & jnp.int32(B3 - 1)
                cnt, _ = plsc.scan_count(d)
                pos = plsc.load_gather(hist1, [g * B3 + d]) + cnt - 1
                plsc.addupdate_scatter(hist1, [g * B3 + d], ones)
                plsc.store_scatter(buf_b, [pos], p & jnp.int32(0x7FFF))

        pltpu.sync_copy(buf_b, out_hbm.at[r])
        return c0

    lax.fori_loop(0, ROWS_PER_W, do_row, 0)


_argsort_desc = functools.partial(
    pl.kernel,
    out_type=jax.ShapeDtypeStruct((N_ROWS, ROW), jnp.int32),
    mesh=plsc.VectorSubcoreMesh(core_axis_name="c", subcore_axis_name="s"),
    scratch_types=[
        pltpu.VMEM((ROW,), jnp.float32),
        pltpu.VMEM((ROW,), jnp.int32),
        pltpu.VMEM((ROW,), jnp.int32),
        pltpu.VMEM((G2 * B2,), jnp.int32),
    ],
    compiler_params=pltpu.CompilerParams(needs_layout_passes=False),
)(_sc_body)


@jax.jit
def kernel(inputs):
    return _argsort_desc(inputs)


# speedup vs baseline: 1.3467x; 1.3467x over previous
import functools

import jax
import jax.numpy as jnp
from jax import lax
from jax.experimental import pallas as pl
from jax.experimental.pallas import tpu as pltpu
from jax.experimental.pallas import tpu_sc as plsc

N_ROWS = 128
ROW = 32768
L = 16
NVEC = ROW // L
NUM_CORES = 2
NUM_SUBCORES = 16
WORKERS = NUM_CORES * NUM_SUBCORES
ROWS_PER_W = N_ROWS // WORKERS


def _clear(hist, nvec, unroll=16):
    zeros = jnp.zeros((L,), jnp.int32)

    @plsc.parallel_loop(0, nvec, unroll=unroll)
    def _(i):
        hist[pl.ds(i * L, L)] = zeros


def _excl_prefix(hist, nvec, unroll=8):
    @plsc.parallel_loop(0, nvec, unroll=unroll, carry=jnp.int32(0))
    def _(i, carry):
        h = hist[pl.ds(i * L, L)]
        inc = plsc.cumsum(h)
        hist[pl.ds(i * L, L)] = inc - h + carry
        return carry + jnp.sum(h)


def _scatter_pass(src_load, digit_fn, payload_fn, store_fn, hist, ones,
                  unroll=4):
    def stage(i):
        x = src_load(i)
        d = digit_fn(x)
        cnt, _ = plsc.scan_count(d)
        return d, cnt, payload_fn(x, i)

    def commit(d, cnt, p):
        pos = plsc.load_gather(hist, [d]) + cnt - 1
        store_fn(pos, p)
        plsc.addupdate_scatter(hist, [d], ones)

    def body(i, carry):
        nxt = stage(i + 1)
        commit(*carry)
        return nxt

    last = lax.fori_loop(0, NVEC - 1, body, stage(0), unroll=unroll)
    commit(*last)


def _sc_body(in_hbm, out_hbm, buf_a, buf_b, hist1, hist2, hist3):
    cid = lax.axis_index("c")
    sid = lax.axis_index("s")
    wid = sid * NUM_CORES + cid
    lane = lax.iota(jnp.int32, L)
    ones = jnp.ones((L,), jnp.int32)

    def do_row(j, c0):
        r = wid * ROWS_PER_W + j
        pltpu.sync_copy(in_hbm.at[r], buf_a)

        _clear(hist1, NVEC)
        _clear(hist2, 512 // L)
        _clear(hist3, 256 // L)

        def hall(i, c):
            v = buf_a[pl.ds(i * L, L)]
            u = plsc.bitcast(v, jnp.int32)
            m = lax.shift_right_arithmetic(u, 31)
            k = u ^ ((m ^ jnp.int32(-1)) & jnp.int32(0x7FFFFFFF))
            buf_a[pl.ds(i * L, L)] = plsc.bitcast(k, jnp.float32)
            plsc.addupdate_scatter(hist1, [k & jnp.int32(0x7FFF)], ones)
            plsc.addupdate_scatter(
                hist2, [lax.shift_right_logical(k, 15) & jnp.int32(0x1FF)],
                ones)
            plsc.addupdate_scatter(
                hist3, [lax.shift_right_logical(k, 24) & jnp.int32(0xFF)],
                ones)
            return c

        plsc.parallel_loop(0, NVEC, unroll=4)(
            lambda i: hall(i, 0) and None)

        _excl_prefix(hist1, NVEC)
        _excl_prefix(hist2, 512 // L)
        _excl_prefix(hist3, 256 // L)

        _scatter_pass(
            src_load=lambda i: plsc.bitcast(buf_a[pl.ds(i * L, L)], jnp.int32),
            digit_fn=lambda k: k & jnp.int32(0x7FFF),
            payload_fn=lambda k, i: (k & jnp.int32(-32768)) | (i * L + lane),
            store_fn=lambda pos, p: plsc.store_scatter(buf_b, [pos], p),
            hist=hist1, ones=ones)

        _scatter_pass(
            src_load=lambda i: buf_b[pl.ds(i * L, L)],
            digit_fn=lambda p: lax.shift_right_logical(p, 15)
            & jnp.int32(0x1FF),
            payload_fn=lambda p, i: p,
            store_fn=lambda pos, p: plsc.store_scatter(
                buf_a, [pos], plsc.bitcast(p, jnp.float32)),
            hist=hist2, ones=ones)

        _scatter_pass(
            src_load=lambda i: plsc.bitcast(buf_a[pl.ds(i * L, L)], jnp.int32),
            digit_fn=lambda p: lax.shift_right_logical(p, 24)
            & jnp.int32(0xFF),
            payload_fn=lambda p, i: p & jnp.int32(0x7FFF),
            store_fn=lambda pos, p: plsc.store_scatter(buf_b, [pos], p),
            hist=hist3, ones=ones)

        pltpu.sync_copy(buf_b, out_hbm.at[r])
        return c0

    lax.fori_loop(0, ROWS_PER_W, do_row, 0)


_argsort_desc = functools.partial(
    pl.kernel,
    out_type=jax.ShapeDtypeStruct((N_ROWS, ROW), jnp.int32),
    mesh=plsc.VectorSubcoreMesh(core_axis_name="c", subcore_axis_name="s"),
    scratch_types=[
        pltpu.VMEM((ROW,), jnp.float32),
        pltpu.VMEM((ROW,), jnp.int32),
        pltpu.VMEM((ROW,), jnp.int32),
        pltpu.VMEM((512,), jnp.int32),
        pltpu.VMEM((256,), jnp.int32),
    ],
    compiler_params=pltpu.CompilerParams(needs_layout_passes=False),
)(_sc_body)


@jax.jit
def kernel(inputs):
    return _argsort_desc(inputs)
